# baseline (speedup 1.0000x reference)
import jax
import jax.numpy as jnp
from jax.experimental import pallas as pl

B, S, H, D, K = 4, 2048, 1024, 64, 1024
R = 2048
_NSUB = 8
_TILES = tuple((i * R // _NSUB, (i + 1) * R // _NSUB) for i in range(_NSUB))
N = B * S


def _onehot_lookup(idx, cb_splits):
    oh = (jax.lax.broadcasted_iota(jnp.int32, (idx.shape[0], K), 1)
          == idx[:, None]).astype(jnp.bfloat16)
    y = jax.lax.dot_general(
        oh, cb_splits, (((1,), (0,)), ((), ())),
        preferred_element_type=jnp.float32)
    return (y[:, :D] + y[:, D:2 * D]) + y[:, 2 * D:]


def _split3(cb):
    hi32 = jax.lax.reduce_precision(cb, 8, 7)
    r1 = cb - hi32
    mid32 = jax.lax.reduce_precision(r1, 8, 7)
    lo32 = r1 - mid32
    return jnp.concatenate([hi32.astype(jnp.bfloat16),
                            mid32.astype(jnp.bfloat16),
                            lo32.astype(jnp.bfloat16)], axis=1)


def _proj_in_kernel(x_ref, w_in_ref, b_in_ref, h_ref):
    h_ref[...] = jax.lax.dot_general(
        x_ref[...], w_in_ref[...], (((1,), (0,)), ((), ())),
        preferred_element_type=jnp.float32) + b_in_ref[...]


def _vq1_kernel(h_ref, rr_ref, cb1_ref, cbs1_ref, c1sq_ref, q1_ref, r2_ref):
    cb1t = cb1_ref[...].T
    cbs1 = cbs1_ref[...]
    c1sq = c1sq_ref[...]
    for lo, hi in _TILES:
        h = h_ref[lo:hi, :]
        cross = jax.lax.dot_general(
            h, cb1t, (((1,), (0,)), ((), ())),
            preferred_element_type=jnp.float32)
        d1 = rr_ref[lo:hi, :] - 2.0 * cross + c1sq
        idx1 = jnp.argmin(d1, axis=-1)
        q1 = _onehot_lookup(idx1, cbs1)
        q1_ref[lo:hi, :] = q1
        r2_ref[lo:hi, :] = h - q1


def _vq2_kernel(r2_ref, rr2_ref, q1_ref, cb2_ref, cbs2_ref, c2sq_ref,
                w_out_ref, b_out_ref, out_ref, q2_ref, l1_ref, l2_ref):
    cb2t = cb2_ref[...].T
    cbs2 = cbs2_ref[...]
    c2sq = c2sq_ref[...]
    w_out = w_out_ref[...]
    b_out = b_out_ref[...]
    l1 = jnp.zeros((1, 1), jnp.float32)
    l2 = jnp.zeros((1, 1), jnp.float32)
    for lo, hi in _TILES:
        r2 = r2_ref[lo:hi, :]
        cross = jax.lax.dot_general(
            r2, cb2t, (((1,), (0,)), ((), ())),
            preferred_element_type=jnp.float32)
        d2 = rr2_ref[lo:hi, :] - 2.0 * cross + c2sq
        idx2 = jnp.argmin(d2, axis=-1)
        q2 = _onehot_lookup(idx2, cbs2)
        q2_ref[lo:hi, :] = q2
        qsum = q1_ref[lo:hi, :] + q2
        out_ref[lo:hi, :] = jax.lax.dot_general(
            qsum, w_out, (((1,), (0,)), ((), ())),
            preferred_element_type=jnp.float32) + b_out
        e2 = q2 - r2
        l1 = l1 + jnp.sum(r2 * r2).reshape(1, 1)
        l2 = l2 + jnp.sum(e2 * e2).reshape(1, 1)

    @pl.when(pl.program_id(0) == 0)
    def _init():
        l1_ref[...] = jnp.zeros_like(l1_ref)
        l2_ref[...] = jnp.zeros_like(l2_ref)

    l1_ref[...] += l1
    l2_ref[...] += l2


_ROW = lambda i: (i, 0)
_FIX = lambda i: (0, 0)


@jax.jit
def kernel(x, W_in, b_in, cb1, cb2, W_out, b_out):
    x2 = x.reshape(N, H)
    cbs1 = _split3(cb1)
    cbs2 = _split3(cb2)
    c1sq = jnp.sum(cb1 ** 2, axis=-1).reshape(1, K)
    c2sq = jnp.sum(cb2 ** 2, axis=-1).reshape(1, K)
    grid = (N // R,)

    h = pl.pallas_call(
        _proj_in_kernel,
        grid=grid,
        in_specs=[
            pl.BlockSpec((R, H), _ROW),
            pl.BlockSpec((H, D), _FIX),
            pl.BlockSpec((1, D), _FIX),
        ],
        out_specs=pl.BlockSpec((R, D), _ROW),
        out_shape=jax.ShapeDtypeStruct((N, D), jnp.float32),
    )(x2, W_in, b_in.reshape(1, D))

    rr1 = jnp.sum(h ** 2, axis=-1, keepdims=True)

    q1, r2 = pl.pallas_call(
        _vq1_kernel,
        grid=grid,
        in_specs=[
            pl.BlockSpec((R, D), _ROW),
            pl.BlockSpec((R, 1), _ROW),
            pl.BlockSpec((K, D), _FIX),
            pl.BlockSpec((K, 3 * D), _FIX),
            pl.BlockSpec((1, K), _FIX),
        ],
        out_specs=[
            pl.BlockSpec((R, D), _ROW),
            pl.BlockSpec((R, D), _ROW),
        ],
        out_shape=[
            jax.ShapeDtypeStruct((N, D), jnp.float32),
            jax.ShapeDtypeStruct((N, D), jnp.float32),
        ],
    )(h, rr1, cb1, cbs1, c1sq)

    rr2 = jnp.sum(r2 ** 2, axis=-1, keepdims=True)

    out, q2, l1, l2 = pl.pallas_call(
        _vq2_kernel,
        grid=grid,
        in_specs=[
            pl.BlockSpec((R, D), _ROW),
            pl.BlockSpec((R, 1), _ROW),
            pl.BlockSpec((R, D), _ROW),
            pl.BlockSpec((K, D), _FIX),
            pl.BlockSpec((K, 3 * D), _FIX),
            pl.BlockSpec((1, K), _FIX),
            pl.BlockSpec((D, H), _FIX),
            pl.BlockSpec((1, H), _FIX),
        ],
        out_specs=[
            pl.BlockSpec((R, H), _ROW),
            pl.BlockSpec((R, D), _ROW),
            pl.BlockSpec((1, 1), _FIX),
            pl.BlockSpec((1, 1), _FIX),
        ],
        out_shape=[
            jax.ShapeDtypeStruct((N, H), jnp.float32),
            jax.ShapeDtypeStruct((N, D), jnp.float32),
            jax.ShapeDtypeStruct((1, 1), jnp.float32),
            jax.ShapeDtypeStruct((1, 1), jnp.float32),
        ],
    )(r2, rr2, q1, cb2, cbs2, c2sq, W_out, b_out.reshape(1, H))

    com = (l1[0, 0] + l2[0, 0]) / (2.0 * N * D)
    return (out.reshape(B, S, H), q1.reshape(B, S, D), q2.reshape(B, S, D),
            com)


# speedup vs baseline: 1.0392x; 1.0392x over previous
import jax
import jax.numpy as jnp
from jax.experimental import pallas as pl
from jax.experimental.pallas import tpu as pltpu

B, S, H, D, K = 4, 2048, 1024, 64, 1024
R = 2048
_NSUB = 4
_TILES = tuple((i * R // _NSUB, (i + 1) * R // _NSUB) for i in range(_NSUB))
N = B * S


def _onehot_lookup(idx, cb_splits):
    oh = (jax.lax.broadcasted_iota(jnp.int32, (idx.shape[0], K), 1)
          == idx[:, None]).astype(jnp.bfloat16)
    y = jax.lax.dot_general(
        oh, cb_splits, (((1,), (0,)), ((), ())),
        preferred_element_type=jnp.float32)
    return (y[:, :D] + y[:, D:2 * D]) + y[:, 2 * D:]


def _split3(cb):
    hi32 = jax.lax.reduce_precision(cb, 8, 7)
    r1 = cb - hi32
    mid32 = jax.lax.reduce_precision(r1, 8, 7)
    lo32 = r1 - mid32
    return jnp.concatenate([hi32.astype(jnp.bfloat16),
                            mid32.astype(jnp.bfloat16),
                            lo32.astype(jnp.bfloat16)], axis=1)


def _proj_in_kernel(x_ref, w_in_ref, b_in_ref, h_ref):
    h_ref[...] = jax.lax.dot_general(
        x_ref[...], w_in_ref[...], (((1,), (0,)), ((), ())),
        preferred_element_type=jnp.float32) + b_in_ref[...]


def _vq1_kernel(h_ref, rr_ref, cb1_ref, cbs1_ref, c1sq_ref, q1_ref, r2_ref):
    cb1t = cb1_ref[...].T
    cbs1 = cbs1_ref[...]
    c1sq = c1sq_ref[...]
    for lo, hi in _TILES:
        h = h_ref[lo:hi, :]
        cross = jax.lax.dot_general(
            h, cb1t, (((1,), (0,)), ((), ())),
            preferred_element_type=jnp.float32)
        d1 = rr_ref[lo:hi, :] - 2.0 * cross + c1sq
        idx1 = jnp.argmin(d1, axis=-1)
        q1 = _onehot_lookup(idx1, cbs1)
        q1_ref[lo:hi, :] = q1
        r2_ref[lo:hi, :] = h - q1


def _vq2_kernel(r2_ref, rr2_ref, q1_ref, cb2_ref, cbs2_ref, c2sq_ref,
                w_out_ref, b_out_ref, out_ref, q2_ref, l1_ref, l2_ref):
    cb2t = cb2_ref[...].T
    cbs2 = cbs2_ref[...]
    c2sq = c2sq_ref[...]
    w_out = w_out_ref[...]
    b_out = b_out_ref[...]
    l1 = jnp.zeros((1, 1), jnp.float32)
    l2 = jnp.zeros((1, 1), jnp.float32)
    for lo, hi in _TILES:
        r2 = r2_ref[lo:hi, :]
        cross = jax.lax.dot_general(
            r2, cb2t, (((1,), (0,)), ((), ())),
            preferred_element_type=jnp.float32)
        d2 = rr2_ref[lo:hi, :] - 2.0 * cross + c2sq
        idx2 = jnp.argmin(d2, axis=-1)
        q2 = _onehot_lookup(idx2, cbs2)
        q2_ref[lo:hi, :] = q2
        qsum = q1_ref[lo:hi, :] + q2
        out_ref[lo:hi, :] = jax.lax.dot_general(
            qsum, w_out, (((1,), (0,)), ((), ())),
            preferred_element_type=jnp.float32) + b_out
        e2 = q2 - r2
        l1 = l1 + jnp.sum(r2 * r2).reshape(1, 1)
        l2 = l2 + jnp.sum(e2 * e2).reshape(1, 1)

    l1_ref[...] = l1.reshape(1, 1, 1)
    l2_ref[...] = l2.reshape(1, 1, 1)


_ROW = lambda i: (i, 0)
_FIX = lambda i: (0, 0)


@jax.jit
def kernel(x, W_in, b_in, cb1, cb2, W_out, b_out):
    x2 = x.reshape(N, H)
    cbs1 = _split3(cb1)
    cbs2 = _split3(cb2)
    c1sq = jnp.sum(cb1 ** 2, axis=-1).reshape(1, K)
    c2sq = jnp.sum(cb2 ** 2, axis=-1).reshape(1, K)
    grid = (N // R,)

    h = pl.pallas_call(
        _proj_in_kernel,
        grid=grid,
        in_specs=[
            pl.BlockSpec((R, H), _ROW),
            pl.BlockSpec((H, D), _FIX),
            pl.BlockSpec((1, D), _FIX),
        ],
        out_specs=pl.BlockSpec((R, D), _ROW),
        out_shape=jax.ShapeDtypeStruct((N, D), jnp.float32),
        compiler_params=pltpu.CompilerParams(
            dimension_semantics=("parallel",)),
    )(x2, W_in, b_in.reshape(1, D))

    rr1 = jnp.sum(h ** 2, axis=-1, keepdims=True)

    q1, r2 = pl.pallas_call(
        _vq1_kernel,
        grid=grid,
        in_specs=[
            pl.BlockSpec((R, D), _ROW),
            pl.BlockSpec((R, 1), _ROW),
            pl.BlockSpec((K, D), _FIX),
            pl.BlockSpec((K, 3 * D), _FIX),
            pl.BlockSpec((1, K), _FIX),
        ],
        out_specs=[
            pl.BlockSpec((R, D), _ROW),
            pl.BlockSpec((R, D), _ROW),
        ],
        out_shape=[
            jax.ShapeDtypeStruct((N, D), jnp.float32),
            jax.ShapeDtypeStruct((N, D), jnp.float32),
        ],
        compiler_params=pltpu.CompilerParams(
            dimension_semantics=("parallel",)),
    )(h, rr1, cb1, cbs1, c1sq)

    rr2 = jnp.sum(r2 ** 2, axis=-1, keepdims=True)

    out, q2, l1, l2 = pl.pallas_call(
        _vq2_kernel,
        grid=grid,
        in_specs=[
            pl.BlockSpec((R, D), _ROW),
            pl.BlockSpec((R, 1), _ROW),
            pl.BlockSpec((R, D), _ROW),
            pl.BlockSpec((K, D), _FIX),
            pl.BlockSpec((K, 3 * D), _FIX),
            pl.BlockSpec((1, K), _FIX),
            pl.BlockSpec((D, H), _FIX),
            pl.BlockSpec((1, H), _FIX),
        ],
        out_specs=[
            pl.BlockSpec((R, H), _ROW),
            pl.BlockSpec((R, D), _ROW),
            pl.BlockSpec((1, 1, 1), lambda i: (i, 0, 0)),
            pl.BlockSpec((1, 1, 1), lambda i: (i, 0, 0)),
        ],
        out_shape=[
            jax.ShapeDtypeStruct((N, H), jnp.float32),
            jax.ShapeDtypeStruct((N, D), jnp.float32),
            jax.ShapeDtypeStruct((N // R, 1, 1), jnp.float32),
            jax.ShapeDtypeStruct((N // R, 1, 1), jnp.float32),
        ],
        compiler_params=pltpu.CompilerParams(
            dimension_semantics=("parallel",)),
    )(r2, rr2, q1, cb2, cbs2, c2sq, W_out, b_out.reshape(1, H))

    com = (jnp.sum(l1) + jnp.sum(l2)) / (2.0 * N * D)
    return (out.reshape(B, S, H), q1.reshape(B, S, D), q2.reshape(B, S, D),
            com)


# speedup vs baseline: 1.0617x; 1.0217x over previous
import jax
import jax.numpy as jnp
from jax.experimental import pallas as pl

B, S, H, D, K = 4, 2048, 1024, 64, 1024
R = 2048
_NSUB = 4
_TILES = tuple((i * R // _NSUB, (i + 1) * R // _NSUB) for i in range(_NSUB))
N = B * S


def _onehot_lookup(idx, cb_splits):
    oh = (jax.lax.broadcasted_iota(jnp.int32, (idx.shape[0], K), 1)
          == idx[:, None]).astype(jnp.bfloat16)
    y = jax.lax.dot_general(
        oh, cb_splits, (((1,), (0,)), ((), ())),
        preferred_element_type=jnp.float32)
    return (y[:, :D] + y[:, D:2 * D]) + y[:, 2 * D:]


def _split3(cb):
    hi32 = jax.lax.reduce_precision(cb, 8, 7)
    r1 = cb - hi32
    mid32 = jax.lax.reduce_precision(r1, 8, 7)
    lo32 = r1 - mid32
    return jnp.concatenate([hi32.astype(jnp.bfloat16),
                            mid32.astype(jnp.bfloat16),
                            lo32.astype(jnp.bfloat16)], axis=1)


def _proj_in_kernel(x_ref, w_in_ref, b_in_ref, h_ref):
    h_ref[...] = jax.lax.dot_general(
        x_ref[...], w_in_ref[...], (((1,), (0,)), ((), ())),
        preferred_element_type=jnp.float32) + b_in_ref[...]


def _vq1_kernel(h_ref, rr_ref, cb1_ref, cbs1_ref, c1sq_ref, q1_ref, r2_ref):
    cb1t = cb1_ref[...].T
    cbs1 = cbs1_ref[...]
    c1sq = c1sq_ref[...]
    for lo, hi in _TILES:
        h = h_ref[lo:hi, :]
        cross = jax.lax.dot_general(
            h, cb1t, (((1,), (0,)), ((), ())),
            preferred_element_type=jnp.float32)
        d1 = rr_ref[lo:hi, :] - 2.0 * cross + c1sq
        idx1 = jnp.argmin(d1, axis=-1)
        q1 = _onehot_lookup(idx1, cbs1)
        q1_ref[lo:hi, :] = q1
        r2_ref[lo:hi, :] = h - q1


def _vq2_kernel(r2_ref, rr2_ref, q1_ref, cb2_ref, cbs2_ref, c2sq_ref,
                w_out_ref, b_out_ref, out_ref, q2_ref, l1_ref, l2_ref):
    cb2t = cb2_ref[...].T
    cbs2 = cbs2_ref[...]
    c2sq = c2sq_ref[...]
    w_out = w_out_ref[...]
    b_out = b_out_ref[...]
    l1 = jnp.zeros((1, 1), jnp.float32)
    l2 = jnp.zeros((1, 1), jnp.float32)
    for lo, hi in _TILES:
        r2 = r2_ref[lo:hi, :]
        cross = jax.lax.dot_general(
            r2, cb2t, (((1,), (0,)), ((), ())),
            preferred_element_type=jnp.float32)
        d2 = rr2_ref[lo:hi, :] - 2.0 * cross + c2sq
        idx2 = jnp.argmin(d2, axis=-1)
        q2 = _onehot_lookup(idx2, cbs2)
        q2_ref[lo:hi, :] = q2
        qsum = q1_ref[lo:hi, :] + q2
        out_ref[lo:hi, :] = jax.lax.dot_general(
            qsum, w_out, (((1,), (0,)), ((), ())),
            preferred_element_type=jnp.float32) + b_out
        e2 = q2 - r2
        l1 = l1 + jnp.sum(r2 * r2).reshape(1, 1)
        l2 = l2 + jnp.sum(e2 * e2).reshape(1, 1)

    @pl.when(pl.program_id(0) == 0)
    def _init():
        l1_ref[...] = jnp.zeros_like(l1_ref)
        l2_ref[...] = jnp.zeros_like(l2_ref)

    l1_ref[...] += l1
    l2_ref[...] += l2


_ROW = lambda i: (i, 0)
_FIX = lambda i: (0, 0)


@jax.jit
def kernel(x, W_in, b_in, cb1, cb2, W_out, b_out):
    x2 = x.reshape(N, H)
    cbs1 = _split3(cb1)
    cbs2 = _split3(cb2)
    c1sq = jnp.sum(cb1 ** 2, axis=-1).reshape(1, K)
    c2sq = jnp.sum(cb2 ** 2, axis=-1).reshape(1, K)
    grid = (N // R,)

    h = pl.pallas_call(
        _proj_in_kernel,
        grid=grid,
        in_specs=[
            pl.BlockSpec((R, H), _ROW),
            pl.BlockSpec((H, D), _FIX),
            pl.BlockSpec((1, D), _FIX),
        ],
        out_specs=pl.BlockSpec((R, D), _ROW),
        out_shape=jax.ShapeDtypeStruct((N, D), jnp.float32),
    )(x2, W_in, b_in.reshape(1, D))

    rr1 = jnp.sum(h ** 2, axis=-1, keepdims=True)

    q1, r2 = pl.pallas_call(
        _vq1_kernel,
        grid=grid,
        in_specs=[
            pl.BlockSpec((R, D), _ROW),
            pl.BlockSpec((R, 1), _ROW),
            pl.BlockSpec((K, D), _FIX),
            pl.BlockSpec((K, 3 * D), _FIX),
            pl.BlockSpec((1, K), _FIX),
        ],
        out_specs=[
            pl.BlockSpec((R, D), _ROW),
            pl.BlockSpec((R, D), _ROW),
        ],
        out_shape=[
            jax.ShapeDtypeStruct((N, D), jnp.float32),
            jax.ShapeDtypeStruct((N, D), jnp.float32),
        ],
    )(h, rr1, cb1, cbs1, c1sq)

    rr2 = jnp.sum(r2 ** 2, axis=-1, keepdims=True)

    out, q2, l1, l2 = pl.pallas_call(
        _vq2_kernel,
        grid=grid,
        in_specs=[
            pl.BlockSpec((R, D), _ROW),
            pl.BlockSpec((R, 1), _ROW),
            pl.BlockSpec((R, D), _ROW),
            pl.BlockSpec((K, D), _FIX),
            pl.BlockSpec((K, 3 * D), _FIX),
            pl.BlockSpec((1, K), _FIX),
            pl.BlockSpec((D, H), _FIX),
            pl.BlockSpec((1, H), _FIX),
        ],
        out_specs=[
            pl.BlockSpec((R, H), _ROW),
            pl.BlockSpec((R, D), _ROW),
            pl.BlockSpec((1, 1), _FIX),
            pl.BlockSpec((1, 1), _FIX),
        ],
        out_shape=[
            jax.ShapeDtypeStruct((N, H), jnp.float32),
            jax.ShapeDtypeStruct((N, D), jnp.float32),
            jax.ShapeDtypeStruct((1, 1), jnp.float32),
            jax.ShapeDtypeStruct((1, 1), jnp.float32),
        ],
    )(r2, rr2, q1, cb2, cbs2, c2sq, W_out, b_out.reshape(1, H))

    com = (l1[0, 0] + l2[0, 0]) / (2.0 * N * D)
    return (out.reshape(B, S, H), q1.reshape(B, S, D), q2.reshape(B, S, D),
            com)
